# Initial kernel scaffold; baseline (speedup 1.0000x reference)
import jax
import jax.numpy as jnp
from jax.experimental import pallas as pl


def kernel(x, edge_index, batch, params):
    raise NotImplementedError("write your pallas kernel here")


# speedup vs baseline: 13.7523x; 13.7523x over previous
import functools

import jax
import jax.numpy as jnp
from jax import lax
from jax.experimental import pallas as pl
from jax.experimental.pallas import tpu as pltpu
from jax.experimental.pallas import tpu_sc as plsc

NC = 2
NS = 16
NW = NC * NS
CHUNK = 125
NBUF = 4


@functools.lru_cache(None)
def _make_scatter(n, chunks_per_worker, w):
    npad = -(-n // (NS * 8)) * (NS * 8)
    stripe = npad // NS
    rounds = chunks_per_worker // NBUF
    mesh = plsc.VectorSubcoreMesh(core_axis_name="c", subcore_axis_name="s")

    @functools.partial(
        pl.kernel,
        mesh=mesh,
        out_type=jax.ShapeDtypeStruct((NC * npad, w), jnp.float32),
        scratch_types=[
            pltpu.VMEM((chunks_per_worker, CHUNK), jnp.int32),
            pltpu.VMEM((chunks_per_worker, CHUNK), jnp.int32),
            pltpu.VMEM((NBUF, CHUNK, w), jnp.float32),
            pltpu.VMEM_SHARED((npad, w), jnp.float32),
            pltpu.SemaphoreType.DMA,
        ],
        compiler_params=pltpu.CompilerParams(use_tc_tiling_on_sc=False),
    )
    def sck(u_hbm, src_hbm, dst_hbm, zer_hbm, out_hbm,
            src_v, dst_v, rows_v, agg_sh, gsem):
        cid = lax.axis_index("c")
        sid = lax.axis_index("s")
        wid = sid * NC + cid
        pltpu.sync_copy(src_hbm.at[wid], src_v)
        pltpu.sync_copy(dst_hbm.at[wid], dst_v)
        pltpu.sync_copy(zer_hbm, agg_sh.at[pl.ds(sid * stripe, stripe)])
        plsc.subcore_barrier()

        def body(r, carry):
            for b in range(NBUF):
                c = r * NBUF + b
                pltpu.async_copy(u_hbm.at[src_v.at[c]], rows_v.at[b], gsem)
            for b in range(NBUF):
                c = r * NBUF + b
                pltpu.make_async_copy(
                    u_hbm.at[src_v.at[c]], rows_v.at[b], gsem).wait()
            for b in range(NBUF):
                c = r * NBUF + b
                pltpu.sync_copy(rows_v.at[b], agg_sh.at[dst_v.at[c]],
                                add=True)
            return carry

        lax.fori_loop(0, rounds, body, 0)
        plsc.subcore_barrier()
        pltpu.sync_copy(agg_sh.at[pl.ds(sid * stripe, stripe)],
                        out_hbm.at[pl.ds(cid * npad + sid * stripe, stripe)])

    return sck


def _padw(w):
    for cand in (32, 64, 80, 96):
        if w <= cand:
            return cand
    raise ValueError(w)


def _scatter_partials(u, src3, dst3):
    n, w = u.shape
    wp = _padw(w)
    if wp != w:
        u = jnp.pad(u, ((0, 0), (0, wp - w)))
    npad = -(-n // (NS * 8)) * (NS * 8)
    zer = jnp.zeros((npad // NS, wp), jnp.float32)
    fn = _make_scatter(n, src3.shape[1], wp)
    out = fn(u, src3, dst3, zer)
    return out[:n], out[npad:npad + n]


@functools.lru_cache(None)
def _mm(n, din, dout):
    def body(x_ref, w_ref, o_ref):
        o_ref[...] = jnp.dot(x_ref[...], w_ref[...],
                             preferred_element_type=jnp.float32,
                     precision=lax.Precision.HIGHEST)
    return pl.pallas_call(
        body, out_shape=jax.ShapeDtypeStruct((n, dout), jnp.float32))


_BR = 2000


@functools.lru_cache(None)
def _conv_combine(n, din, dout, apply_rel, relu, next_dout):
    nb = n // _BR
    pw = din if apply_rel else dout

    def body(*refs):
        refs = list(refs)
        p0_ref, p1_ref, z_ref = refs[:3]
        refs = refs[3:]
        if apply_rel:
            wrel_ref = refs.pop(0)
        brel_ref, wroot_ref, g_ref, b_ref = refs[:4]
        refs = refs[4:]
        if next_dout:
            wnext_ref = refs.pop(0)
        out_ref = refs.pop(0)
        u_ref = refs.pop(0) if next_dout else None
        y_sc, st_sc = refs
        ph = pl.program_id(0)
        i = pl.program_id(1)

        @pl.when(ph == 0)
        def _phase0():
            agg = p0_ref[...] + p1_ref[...]
            if apply_rel:
                agg = jnp.dot(agg, wrel_ref[...],
                              preferred_element_type=jnp.float32,
                              precision=lax.Precision.HIGHEST)
            y = agg + brel_ref[...] + jnp.dot(
                z_ref[...], wroot_ref[...],
                preferred_element_type=jnp.float32,
                precision=lax.Precision.HIGHEST)
            y_sc[pl.ds(i * _BR, _BR), :] = y
            s = jnp.concatenate(
                [jnp.sum(y, 0, keepdims=True),
                 jnp.sum(y * y, 0, keepdims=True)], axis=0)

            @pl.when(i == 0)
            def _():
                st_sc[...] = s

            @pl.when(i > 0)
            def _():
                st_sc[...] = st_sc[...] + s

        @pl.when(ph == 1)
        def _phase1():
            y = y_sc[pl.ds(i * _BR, _BR), :]
            m = st_sc[0:1] * (1.0 / n)
            v = st_sc[1:2] * (1.0 / n) - m * m
            yn = (y - m) * lax.rsqrt(v + 1e-5) * g_ref[...] + b_ref[...]
            if relu:
                yn = jnp.maximum(yn, 0.0)
            out_ref[...] = yn
            if next_dout:
                u_ref[...] = jnp.dot(yn, wnext_ref[...],
                                     preferred_element_type=jnp.float32,
                                     precision=lax.Precision.HIGHEST)

    def row_in(p, i):
        return (jnp.where(p == 0, i, 0), 0)

    def row_out(p, i):
        return (jnp.where(p == 0, 0, i), 0)

    full = lambda p, i: (0, 0)
    in_specs = [pl.BlockSpec((_BR, pw), row_in),
                pl.BlockSpec((_BR, pw), row_in),
                pl.BlockSpec((_BR, din), row_in)]
    if apply_rel:
        in_specs.append(pl.BlockSpec((din, dout), full))
    in_specs += [pl.BlockSpec((1, dout), full),
                 pl.BlockSpec((din, dout), full),
                 pl.BlockSpec((1, dout), full),
                 pl.BlockSpec((1, dout), full)]
    out_specs = [pl.BlockSpec((_BR, dout), row_out)]
    outs = [jax.ShapeDtypeStruct((n, dout), jnp.float32)]
    if next_dout:
        in_specs.append(pl.BlockSpec((dout, next_dout), full))
        out_specs.append(pl.BlockSpec((_BR, next_dout), row_out))
        outs.append(jax.ShapeDtypeStruct((n, next_dout), jnp.float32))
    if len(outs) == 1:
        outs, out_specs = outs[0], out_specs[0]
    return pl.pallas_call(
        body,
        grid=(2, nb),
        in_specs=in_specs,
        out_specs=out_specs,
        out_shape=outs,
        scratch_shapes=[pltpu.VMEM((n, dout), jnp.float32),
                        pltpu.VMEM((2, dout), jnp.float32)],
    )


@functools.lru_cache(None)
def _pool_proj(n, dh, ng, dout):
    def body(h_ref, b_ref, wp_ref, bp_ref, o_ref):
        h = h_ref[...]
        b = b_ref[...]
        gid = lax.broadcasted_iota(jnp.int32, (ng, 1), 0)
        oh = (b == gid).astype(jnp.float32)
        sums = jnp.dot(oh, h, preferred_element_type=jnp.float32,
                     precision=lax.Precision.HIGHEST)
        cnt = jnp.sum(oh, axis=1, keepdims=True)
        pooled = sums / jnp.maximum(cnt, 1.0)
        o_ref[...] = jnp.dot(pooled, wp_ref[...],
                             preferred_element_type=jnp.float32,
                     precision=lax.Precision.HIGHEST) + bp_ref[...]
    return pl.pallas_call(
        body, out_shape=jax.ShapeDtypeStruct((ng, dout), jnp.float32))


def _row(v):
    return v.reshape(1, -1)


def _combine(P_pieces, widths, z, conv, bn, relu, apply_rel, wnext=None):
    n = z.shape[0]
    P0 = jnp.concatenate(
        [p[0][:, :wr] for p, wr in zip(P_pieces, widths)], axis=1) \
        if len(P_pieces) > 1 else P_pieces[0][0][:, :widths[0]]
    P1 = jnp.concatenate(
        [p[1][:, :wr] for p, wr in zip(P_pieces, widths)], axis=1) \
        if len(P_pieces) > 1 else P_pieces[0][1][:, :widths[0]]
    args = [P0, P1, z]
    if apply_rel:
        args.append(conv['Wrel'])
    args += [_row(conv['brel']), conv['Wroot'], _row(bn['gamma']),
             _row(bn['beta'])]
    if wnext is not None:
        args.append(wnext)
    dout = conv['Wroot'].shape[1]
    fn = _conv_combine(n, z.shape[1], dout, apply_rel, relu,
                       wnext.shape[1] if wnext is not None else 0)
    return fn(*args)


def kernel(x, edge_index, batch, params):
    n = x.shape[0]
    e = edge_index.shape[1]
    cpw = e // NW // CHUNK
    src3 = edge_index[0].reshape(NW, cpw, CHUNK)
    dst3 = edge_index[1].reshape(NW, cpw, CHUNK)

    def scat(u):
        return _scatter_partials(u, src3, dst3)

    ic = params['init_conv']
    u1 = _mm(n, x.shape[1], ic['Wrel'].shape[1])(x, ic['Wrel'])
    s1 = scat(u1)
    h = _combine([s1], [u1.shape[1]], x, ic, params['init_bn'],
                 relu=False, apply_rel=False)

    for blk in params['blocks']:
        Ps = [scat(h)]
        widths = [h.shape[1]]
        xs = [h]
        for lyr in blk['layers']:
            z = jnp.concatenate(xs, axis=1) if len(xs) > 1 else xs[0]
            z1, u2 = _combine(Ps, widths, z, lyr['conv1'], lyr['bn1'],
                              relu=True, apply_rel=True,
                              wnext=lyr['conv2']['Wrel'])
            s2 = scat(u2)
            z2 = _combine([s2], [u2.shape[1]], z1, lyr['conv2'], lyr['bn2'],
                          relu=True, apply_rel=False)
            Ps.append(scat(z2))
            widths.append(z2.shape[1])
            xs.append(z2)
        zc = jnp.concatenate(xs, axis=1)
        h = _combine(Ps, widths, zc, blk['trans_conv'], blk['trans_bn'],
                     relu=False, apply_rel=True)

    pr = params['proj']
    return _pool_proj(n, h.shape[1], 64, pr['W'].shape[1])(
        h, batch.reshape(1, n).astype(jnp.int32), pr['W'], _row(pr['b']))
